# baseline (speedup 1.0000x reference)
import functools

import jax
import jax.numpy as jnp
from jax import lax
from jax.experimental import pallas as pl
from jax.experimental.pallas import tpu as pltpu
from jax.experimental.pallas import tpu_sc as plsc

_LANES = 16


def _sc_fused_body(toks_per_w, n_used, L, stab_id_hbm, cycle_id_hbm, stab_hbm,
                   cyc_hbm, val_hbm, fused_hbm, sidx, cidx, srows, crows,
                   valv, f1, sem):
    D = srows.shape[1]
    wid = lax.axis_index("s") * 2 + lax.axis_index("c")

    @pl.when(wid < n_used)
    def _():
        tok0 = wid * toks_per_w
        cpi1 = pltpu.async_copy(stab_id_hbm.at[pl.ds(tok0, toks_per_w)],
                                sidx, sem)
        cpi2 = pltpu.async_copy(cycle_id_hbm.at[pl.ds(tok0, toks_per_w)],
                                cidx, sem)
        cpi1.wait()
        cpi2.wait()
        pltpu.sync_copy(val_hbm, valv)
        cpg1 = pltpu.async_copy(stab_hbm.at[sidx], srows, sem)
        cpg2 = pltpu.async_copy(cyc_hbm.at[cidx], crows, sem)
        cpg1.wait()
        cpg2.wait()
        for j in range(toks_per_w):
            for k in range(D // _LANES):
                ds = pl.ds(k * _LANES, _LANES)
                s = srows[j, ds] + crows[j, ds]
                f1[j, ds] = s + valv[1, ds]
                srows[j, ds] = s + valv[0, ds]
        pltpu.sync_copy(srows, fused_hbm.at[pl.ds(tok0, toks_per_w)])
        pltpu.sync_copy(f1, fused_hbm.at[pl.ds(L + tok0, toks_per_w)])


def _sc_stream_body(bpw, L, Lp, half, syn_hbm, fused_hbm, out_hbm,
                    synv, idxa, idxb, rowsa, rowsb, sem):
    D = rowsa.shape[1]
    wid = lax.axis_index("s") * 2 + lax.axis_index("c")
    b0 = wid * bpw

    def per_batch(i, _):
        bglob = b0 + i
        pltpu.sync_copy(syn_hbm.at[bglob], synv)
        for k in range(Lp // _LANES):
            ds = pl.ds(k * _LANES, _LANES)
            idx = (synv[ds] * L
                   + lax.broadcasted_iota(jnp.int32, (_LANES,), 0)
                   + k * _LANES)
            idx = jnp.minimum(idx, 2 * L - 1)
            if k < half // _LANES:
                idxa[pl.ds(k * _LANES, _LANES)] = idx
            else:
                idxb[pl.ds(k * _LANES - half, _LANES)] = idx
        cpa = pltpu.async_copy(fused_hbm.at[idxa], rowsa, sem)
        cpb = pltpu.async_copy(fused_hbm.at[idxb], rowsb, sem)
        cpa.wait()
        cpb.wait()
        pltpu.sync_copy(rowsa, out_hbm.at[bglob, pl.ds(0, half)])
        pltpu.sync_copy(rowsb.at[pl.ds(0, L - half)],
                        out_hbm.at[bglob, pl.ds(half, L - half)])
        return 0

    lax.fori_loop(0, bpw, per_batch, 0)


def _sc_build_fused(stab_id, cycle_id, stab_table, cycle_table, val_table):
    L = stab_id.shape[0]
    D = stab_table.shape[1]
    toks_per_w = 8
    n_used = L // toks_per_w
    mesh = plsc.VectorSubcoreMesh(core_axis_name="c", subcore_axis_name="s")
    body = functools.partial(_sc_fused_body, toks_per_w, n_used, L)
    return pl.kernel(
        body,
        out_type=jax.ShapeDtypeStruct((2 * L, D), jnp.float32),
        mesh=mesh,
        scratch_types=[
            pltpu.VMEM((toks_per_w,), jnp.int32),
            pltpu.VMEM((toks_per_w,), jnp.int32),
            pltpu.VMEM((toks_per_w, D), jnp.float32),
            pltpu.VMEM((toks_per_w, D), jnp.float32),
            pltpu.VMEM((2, D), jnp.float32),
            pltpu.VMEM((toks_per_w, D), jnp.float32),
            pltpu.SemaphoreType.DMA,
        ],
    )(stab_id, cycle_id, stab_table, cycle_table, val_table)


def kernel(syndrome, stab_id, cycle_id, stab_table, cycle_table, val_table):
    B, L = syndrome.shape
    D = stab_table.shape[1]
    bpw = B // 32
    Lp = 224
    half = 112

    fused = _sc_build_fused(stab_id.astype(jnp.int32),
                            cycle_id.astype(jnp.int32),
                            stab_table, cycle_table, val_table)
    syn = jnp.pad(syndrome.astype(jnp.int32), ((0, 0), (0, Lp - L)))

    mesh = plsc.VectorSubcoreMesh(core_axis_name="c", subcore_axis_name="s")
    body = functools.partial(_sc_stream_body, bpw, L, Lp, half)
    return pl.kernel(
        body,
        out_type=jax.ShapeDtypeStruct((B, L, D), jnp.float32),
        mesh=mesh,
        scratch_types=[
            pltpu.VMEM((Lp,), jnp.int32),
            pltpu.VMEM((half,), jnp.int32),
            pltpu.VMEM((Lp - half,), jnp.int32),
            pltpu.VMEM((half, D), jnp.float32),
            pltpu.VMEM((Lp - half, D), jnp.float32),
            pltpu.SemaphoreType.DMA,
        ],
    )(syn, fused)


# speedup vs baseline: 3.8988x; 3.8988x over previous
import functools

import jax
import jax.numpy as jnp
from jax import lax
from jax.experimental import pallas as pl
from jax.experimental.pallas import tpu as pltpu
from jax.experimental.pallas import tpu_sc as plsc

_LANES = 16


def _sc_base_body(toks_per_w, n_used, stab_id_hbm, cycle_id_hbm, stab_hbm,
                  cyc_hbm, base_hbm, sidx, cidx, srows, crows, sem):
    D = srows.shape[1]
    wid = lax.axis_index("s") * 2 + lax.axis_index("c")

    @pl.when(wid < n_used)
    def _():
        tok0 = wid * toks_per_w
        cpi1 = pltpu.async_copy(stab_id_hbm.at[pl.ds(tok0, toks_per_w)],
                                sidx, sem)
        cpi2 = pltpu.async_copy(cycle_id_hbm.at[pl.ds(tok0, toks_per_w)],
                                cidx, sem)
        cpi1.wait()
        cpi2.wait()
        cpg1 = pltpu.async_copy(stab_hbm.at[sidx], srows, sem)
        cpg2 = pltpu.async_copy(cyc_hbm.at[cidx], crows, sem)
        cpg1.wait()
        cpg2.wait()
        for j in range(toks_per_w):
            for k in range(D // _LANES):
                ds = pl.ds(k * _LANES, _LANES)
                srows[j, ds] = srows[j, ds] + crows[j, ds]
        pltpu.sync_copy(srows, base_hbm.at[pl.ds(tok0, toks_per_w)])


def _sc_build_base(stab_id, cycle_id, stab_table, cycle_table):
    L = stab_id.shape[0]
    D = stab_table.shape[1]
    toks_per_w = 8
    n_used = L // toks_per_w
    mesh = plsc.VectorSubcoreMesh(core_axis_name="c", subcore_axis_name="s")
    body = functools.partial(_sc_base_body, toks_per_w, n_used)
    return pl.kernel(
        body,
        out_type=jax.ShapeDtypeStruct((L, D), jnp.float32),
        mesh=mesh,
        scratch_types=[
            pltpu.VMEM((toks_per_w,), jnp.int32),
            pltpu.VMEM((toks_per_w,), jnp.int32),
            pltpu.VMEM((toks_per_w, D), jnp.float32),
            pltpu.VMEM((toks_per_w, D), jnp.float32),
            pltpu.SemaphoreType.DMA,
        ],
    )(stab_id, cycle_id, stab_table, cycle_table)


def _tc_stream_body(syn_ref, base_ref, val_ref, out_ref):
    syn = syn_ref[...].astype(jnp.float32)
    b0 = base_ref[...] + val_ref[0, :][None, :]
    diff = val_ref[1, :] - val_ref[0, :]
    out_ref[...] = b0[None, :, :] + syn[:, :, None] * diff[None, None, :]


def kernel(syndrome, stab_id, cycle_id, stab_table, cycle_table, val_table):
    B, L = syndrome.shape
    D = stab_table.shape[1]
    BB = 32

    base = _sc_build_base(stab_id.astype(jnp.int32),
                          cycle_id.astype(jnp.int32),
                          stab_table, cycle_table)
    syn = syndrome.astype(jnp.int32)

    return pl.pallas_call(
        _tc_stream_body,
        grid=(B // BB,),
        in_specs=[
            pl.BlockSpec((BB, L), lambda i: (i, 0)),
            pl.BlockSpec((L, D), lambda i: (0, 0)),
            pl.BlockSpec((2, D), lambda i: (0, 0)),
        ],
        out_specs=pl.BlockSpec((BB, L, D), lambda i: (i, 0, 0)),
        out_shape=jax.ShapeDtypeStruct((B, L, D), jnp.float32),
    )(syn, base, val_table)
